# Initial kernel scaffold; baseline (speedup 1.0000x reference)
import jax
import jax.numpy as jnp
from jax.experimental import pallas as pl


def kernel(x, vec, edge_index, r_ij, f_ij, d_ij, batch, ln_w, ln_b, vec_ln_w, Wq, Wk, Wv, vec_proj_W, dv_proj_W, dv_proj_b, s_proj_W, s_proj_b, o_proj_W, o_proj_b):
    raise NotImplementedError("write your pallas kernel here")


# speedup vs baseline: 5.1650x; 5.1650x over previous
import functools

import jax
import jax.numpy as jnp
from jax import lax
from jax.experimental import pallas as pl
from jax.experimental.pallas import tpu as pltpu
from jax.experimental.pallas import tpu_sc as plsc

N = 10000
E = 160000
H = 128
NH = 8
HD = H // NH
B = 100
NPS = N // B
VDIM = 8
CUTOFF = 5.0

NC = 2
NS = 16
NW = NC * NS

N_OUT = 10240
ACC2 = 10496
TRASH = N_OUT
ZR = ACC2 // NS
WR = N_OUT // NS

EPT = 10240
E_PAD = EPT * NS
GB = 128
NBB = EPT // GB
GPW = E_PAD // NW
GCH = 128
GCH2 = 80


def _silu(x):
    return x * jax.nn.sigmoid(x)


def _attn_kernel(x_ref, lnw_ref, lnb_ref, wq_ref, wk_ref, wv_ref, v_ref):
    xb = x_ref[0]
    m = jnp.mean(xb, axis=-1, keepdims=True)
    var = jnp.mean((xb - m) ** 2, axis=-1, keepdims=True)
    xb = (xb - m) / jnp.sqrt(var + 1e-5) * lnw_ref[0] + lnb_ref[0]
    q = jnp.dot(xb, wq_ref[...], preferred_element_type=jnp.float32)
    k = jnp.dot(xb, wk_ref[...], preferred_element_type=jnp.float32)
    vv = jnp.dot(xb, wv_ref[...], preferred_element_type=jnp.float32)
    scale = 1.0 / (HD ** 0.5)
    for h in range(NH):
        qh = q[:, h * HD:(h + 1) * HD]
        kh = k[:, h * HD:(h + 1) * HD]
        vh = vv[:, h * HD:(h + 1) * HD]
        sc = lax.dot_general(qh, kh, (((1,), (1,)), ((), ())),
                             preferred_element_type=jnp.float32) * scale
        sc = sc - jnp.max(sc, axis=-1, keepdims=True)
        ex = jnp.exp(sc)
        at = ex / jnp.sum(ex, axis=-1, keepdims=True)
        v3 = lax.dot_general(at, vh, (((1,), (0,)), ((), ())),
                             preferred_element_type=jnp.float32)
        v_ref[0, :, h * HD:(h + 1) * HD] = v3 * (1.0 / NPS)


def _attention(x, ln_w, ln_b, Wq, Wk, Wv):
    xg = x.reshape(B, NPS, H)
    out = pl.pallas_call(
        _attn_kernel,
        grid=(B,),
        in_specs=[
            pl.BlockSpec((1, NPS, H), lambda b: (b, 0, 0)),
            pl.BlockSpec((1, H), lambda b: (0, 0)),
            pl.BlockSpec((1, H), lambda b: (0, 0)),
            pl.BlockSpec((H, H), lambda b: (0, 0)),
            pl.BlockSpec((H, H), lambda b: (0, 0)),
            pl.BlockSpec((H, H), lambda b: (0, 0)),
        ],
        out_specs=pl.BlockSpec((1, NPS, H), lambda b: (b, 0, 0)),
        out_shape=jax.ShapeDtypeStruct((B, NPS, H), jnp.float32),
    )(xg, ln_w.reshape(1, H), ln_b.reshape(1, H), Wq, Wk, Wv)
    return out.reshape(N, H)


def _vec_kernel(vec_ref, w1_ref, w3_ref, vec3_ref, vsum_ref):
    acc = jnp.zeros((vec_ref.shape[1], H), jnp.float32)
    for d in range(VDIM):
        vd = vec_ref[0, :, d, :]
        acc = acc + vd
        vec3_ref[0, :, d, :] = jnp.dot(vd, w3_ref[...],
                                       preferred_element_type=jnp.float32)
    vsum_ref[0] = jnp.dot(acc, w1_ref[...], preferred_element_type=jnp.float32)


def _vec_proj(vec, vec_proj_W, vec_ln_w):
    wf = vec_proj_W * vec_ln_w[:, None]
    w1 = wf[:, :H]
    w3 = wf[:, H:]
    CH = 1000
    nch = N // CH
    vec3, vsum = pl.pallas_call(
        _vec_kernel,
        grid=(nch,),
        in_specs=[
            pl.BlockSpec((1, CH, VDIM, H), lambda i: (i, 0, 0, 0)),
            pl.BlockSpec((H, H), lambda i: (0, 0)),
            pl.BlockSpec((H, H), lambda i: (0, 0)),
        ],
        out_specs=[
            pl.BlockSpec((1, CH, VDIM, H), lambda i: (i, 0, 0, 0)),
            pl.BlockSpec((1, CH, H), lambda i: (i, 0, 0)),
        ],
        out_shape=[
            jax.ShapeDtypeStruct((nch, CH, VDIM, H), jnp.float32),
            jax.ShapeDtypeStruct((nch, CH, H), jnp.float32),
        ],
    )(vec.reshape(nch, CH, VDIM, H), w1, w3)
    return vec3.reshape(N, VDIM, H), vsum.reshape(N, H)


def _dv_kernel(f_ref, r_ref, w_ref, b_ref, dv_ref):
    pre = jnp.dot(f_ref[...], w_ref[...],
                  preferred_element_type=jnp.float32) + b_ref[0]
    r = r_ref[...]
    cut = 0.5 * (jnp.cos(jnp.pi * r / CUTOFF) + 1.0)
    cut = jnp.where(r < CUTOFF, cut, 0.0)
    dv_ref[...] = _silu(pre) * cut


def _dv(f_pad, r_pad, dv_proj_W, dv_proj_b):
    CH = 4096
    nch = E_PAD // CH
    return pl.pallas_call(
        _dv_kernel,
        grid=(nch,),
        in_specs=[
            pl.BlockSpec((CH, H), lambda i: (i, 0)),
            pl.BlockSpec((CH, 1), lambda i: (i, 0)),
            pl.BlockSpec((H, H), lambda i: (0, 0)),
            pl.BlockSpec((1, H), lambda i: (0, 0)),
        ],
        out_specs=pl.BlockSpec((CH, H), lambda i: (i, 0)),
        out_shape=jax.ShapeDtypeStruct((E_PAD, H), jnp.float32),
    )(f_pad, r_pad, dv_proj_W, dv_proj_b.reshape(1, H))


def _sc_gather_body(v_hbm, src_hbm, out_hbm, idx_v, rows_v, sem):
    wid = lax.axis_index("c") * NS + lax.axis_index("s")
    base = wid * GPW
    for ch in range(GPW // GCH):
        off = base + ch * GCH
        pltpu.sync_copy(src_hbm.at[pl.ds(off, GCH)], idx_v)
        pltpu.async_copy(v_hbm.at[idx_v], rows_v, sem).wait()
        pltpu.sync_copy(rows_v, out_hbm.at[pl.ds(off, GCH)])


def _sc_gather(v, src_pad):
    mesh = plsc.VectorSubcoreMesh(core_axis_name="c", subcore_axis_name="s")
    f = pl.kernel(
        _sc_gather_body,
        out_type=jax.ShapeDtypeStruct((E_PAD, H), jnp.float32),
        mesh=mesh,
        scratch_types=[
            pltpu.VMEM((GCH,), jnp.int32),
            pltpu.VMEM((GCH, H), jnp.float32),
            pltpu.SemaphoreType.DMA,
        ],
    )
    return f(v, src_pad)


def _edge_kernel(vj_ref, dv_ref, w_ref, b_ref, lnv_ref, m_ref, s1_ref, s2_ref):
    m = vj_ref[...] * dv_ref[...]
    s = _silu(jnp.dot(m, w_ref[...],
                      preferred_element_type=jnp.float32) + b_ref[0])
    m_ref[...] = m
    s1_ref[...] = s[:, :H] * lnv_ref[0]
    s2_ref[...] = s[:, H:]


def _edge(vj0, dvp, s_proj_W, s_proj_b, vec_ln_w):
    CH = 4096
    nch = E_PAD // CH
    return pl.pallas_call(
        _edge_kernel,
        grid=(nch,),
        in_specs=[
            pl.BlockSpec((CH, H), lambda i: (i, 0)),
            pl.BlockSpec((CH, H), lambda i: (i, 0)),
            pl.BlockSpec((H, 2 * H), lambda i: (0, 0)),
            pl.BlockSpec((1, 2 * H), lambda i: (0, 0)),
            pl.BlockSpec((1, H), lambda i: (0, 0)),
        ],
        out_specs=[
            pl.BlockSpec((CH, H), lambda i: (i, 0)),
            pl.BlockSpec((CH, H), lambda i: (i, 0)),
            pl.BlockSpec((CH, H), lambda i: (i, 0)),
        ],
        out_shape=[
            jax.ShapeDtypeStruct((E_PAD, H), jnp.float32),
            jax.ShapeDtypeStruct((E_PAD, H), jnp.float32),
            jax.ShapeDtypeStruct((E_PAD, H), jnp.float32),
        ],
    )(vj0, dvp, s_proj_W, s_proj_b.reshape(1, 2 * H), vec_ln_w.reshape(1, H))


def _sc_gather_vec_body(vec_hbm, src_hbm, out_hbm, idx_v, rows_v, sem):
    wid = lax.axis_index("c") * NS + lax.axis_index("s")
    base = wid * GPW
    for ch in range(GPW // GCH2):
        off = base + ch * GCH2
        pltpu.sync_copy(src_hbm.at[pl.ds(off, GCH2)], idx_v)
        pltpu.async_copy(vec_hbm.at[idx_v], rows_v, sem).wait()
        pltpu.sync_copy(rows_v, out_hbm.at[pl.ds(off, GCH2)])


def _sc_gather_vec(vec2d, src_pad):
    mesh = plsc.VectorSubcoreMesh(core_axis_name="c", subcore_axis_name="s")
    f = pl.kernel(
        _sc_gather_vec_body,
        out_type=jax.ShapeDtypeStruct((E_PAD, VDIM * H), jnp.float32),
        mesh=mesh,
        scratch_types=[
            pltpu.VMEM((GCH2,), jnp.int32),
            pltpu.VMEM((GCH2, VDIM * H), jnp.float32),
            pltpu.SemaphoreType.DMA,
        ],
    )
    return f(vec2d, src_pad)


def _vecj_kernel(vj0_ref, s1_ref, s2_ref, dp_ref, out_ref):
    s1 = s1_ref[...]
    s2 = s2_ref[...]
    for d in range(VDIM):
        out_ref[d] = (vj0_ref[:, d, :] * s1
                      + s2 * dp_ref[:, d:d + 1])


def _vecj(vecj0, s1, s2, d_ij):
    CHE = 2048
    nch = E_PAD // CHE
    dp = jnp.pad(d_ij, ((0, E_PAD - E), (0, 0)))
    return pl.pallas_call(
        _vecj_kernel,
        grid=(nch,),
        in_specs=[
            pl.BlockSpec((CHE, VDIM, H), lambda i: (i, 0, 0)),
            pl.BlockSpec((CHE, H), lambda i: (i, 0)),
            pl.BlockSpec((CHE, H), lambda i: (i, 0)),
            pl.BlockSpec((CHE, VDIM), lambda i: (i, 0)),
        ],
        out_specs=pl.BlockSpec((VDIM, CHE, H), lambda i: (0, i, 0)),
        out_shape=jax.ShapeDtypeStruct((VDIM, E_PAD, H), jnp.float32),
    )(vecj0.reshape(E_PAD, VDIM, H), s1, s2, dp)


def _sc_scatter_body(m_hbm, vjT_hbm, dst_hbm, z_hbm, xagg_hbm, vagg_hbm,
                     dstraw, idxbuf, databuf, acc):
    c = lax.axis_index("c")
    sid = lax.axis_index("s")

    def do_slab(src4d, slab_out):
        pltpu.sync_copy(z_hbm, acc.at[pl.ds(sid * ZR, ZR)])
        plsc.subcore_barrier()

        def batch(b, carry):
            pltpu.sync_copy(dst_hbm.at[sid, b], dstraw)
            for j in range(GB // 16):
                d16 = dstraw[pl.ds(j * 16, 16)]
                idxbuf[pl.ds(j * 16, 16)] = jnp.where(d16 < 0, TRASH, d16)
            pltpu.sync_copy(src4d.at[sid, b], databuf)
            pltpu.sync_copy(databuf, acc.at[idxbuf], add=True)
            return carry

        lax.fori_loop(0, NBB, batch, jnp.int32(0))
        plsc.subcore_barrier()
        pltpu.sync_copy(acc.at[pl.ds(sid * WR, WR)],
                        slab_out.at[pl.ds(sid * WR, WR)])
        plsc.subcore_barrier()

    @pl.when(c == 0)
    def _():
        do_slab(m_hbm, xagg_hbm)
        for d in (0, 1, 2):
            do_slab(vjT_hbm.at[d], vagg_hbm.at[d])

    @pl.when(c == 1)
    def _():
        for d in (3, 4, 5, 6, 7):
            do_slab(vjT_hbm.at[d], vagg_hbm.at[d])


def _sc_scatter(m, vecjT, dst_pad):
    mesh = plsc.VectorSubcoreMesh(core_axis_name="c", subcore_axis_name="s")
    f = pl.kernel(
        _sc_scatter_body,
        out_type=[
            jax.ShapeDtypeStruct((N_OUT, H), jnp.float32),
            jax.ShapeDtypeStruct((VDIM, N_OUT, H), jnp.float32),
        ],
        mesh=mesh,
        scratch_types=[
            pltpu.VMEM((GB,), jnp.int32),
            pltpu.VMEM((GB,), jnp.int32),
            pltpu.VMEM((GB, H), jnp.float32),
            pltpu.VMEM_SHARED((ACC2, H), jnp.float32),
        ],
    )
    z = jnp.zeros((ZR, H), jnp.float32)
    xagg, vaggT = f(m.reshape(NS, NBB, GB, H),
                    vecjT.reshape(VDIM, NS, NBB, GB, H),
                    dst_pad.reshape(NS, NBB, GB), z)
    return xagg, vaggT


def _final_kernel(xagg_ref, vsum_ref, vec3_ref, vagg_ref, w_ref, b_ref,
                  dx_ref, dvec_ref):
    o = jnp.dot(xagg_ref[0], w_ref[...],
                preferred_element_type=jnp.float32) + b_ref[0]
    o1 = o[:, :H]
    o2 = o[:, H:2 * H]
    o3 = o[:, 2 * H:]
    dx_ref[0] = vsum_ref[0] * o2 + o3
    for d in range(VDIM):
        dvec_ref[0, :, d, :] = vec3_ref[0, :, d, :] * o1 + vagg_ref[0, :, d, :]


def _final(xagg, vsum, vec3, vagg, o_proj_W, o_proj_b):
    CH = 1000
    nch = N // CH
    dx, dvec = pl.pallas_call(
        _final_kernel,
        grid=(nch,),
        in_specs=[
            pl.BlockSpec((1, CH, H), lambda i: (i, 0, 0)),
            pl.BlockSpec((1, CH, H), lambda i: (i, 0, 0)),
            pl.BlockSpec((1, CH, VDIM, H), lambda i: (i, 0, 0, 0)),
            pl.BlockSpec((1, CH, VDIM, H), lambda i: (i, 0, 0, 0)),
            pl.BlockSpec((H, 3 * H), lambda i: (0, 0)),
            pl.BlockSpec((1, 3 * H), lambda i: (0, 0)),
        ],
        out_specs=[
            pl.BlockSpec((1, CH, H), lambda i: (i, 0, 0)),
            pl.BlockSpec((1, CH, VDIM, H), lambda i: (i, 0, 0, 0)),
        ],
        out_shape=[
            jax.ShapeDtypeStruct((nch, CH, H), jnp.float32),
            jax.ShapeDtypeStruct((nch, CH, VDIM, H), jnp.float32),
        ],
    )(xagg.reshape(nch, CH, H), vsum.reshape(nch, CH, H),
      vec3.reshape(nch, CH, VDIM, H), vagg.reshape(nch, CH, VDIM, H),
      o_proj_W, o_proj_b.reshape(1, 3 * H))
    return dx.reshape(N, H), dvec.reshape(N, VDIM, H)


def kernel(x, vec, edge_index, r_ij, f_ij, d_ij, batch, ln_w, ln_b, vec_ln_w,
           Wq, Wk, Wv, vec_proj_W, dv_proj_W, dv_proj_b,
           s_proj_W, s_proj_b, o_proj_W, o_proj_b):
    del batch

    src = edge_index[0]
    dst = edge_index[1]
    pad_e = E_PAD - E
    src_pad = jnp.pad(src, (0, pad_e))
    dst_pad = jnp.pad(dst, (0, pad_e), constant_values=-1)
    f_pad = jnp.pad(f_ij, ((0, pad_e), (0, 0)))
    r_pad = jnp.pad(r_ij, (0, pad_e), constant_values=2.0 * CUTOFF)

    v = _attention(x, ln_w, ln_b, Wq, Wk, Wv)
    vec3, vsum = _vec_proj(vec, vec_proj_W, vec_ln_w)
    dvp = _dv(f_pad, r_pad.reshape(E_PAD, 1), dv_proj_W, dv_proj_b)

    vj0 = _sc_gather(v, src_pad)
    m, s1, s2 = _edge(vj0, dvp, s_proj_W, s_proj_b, vec_ln_w)

    vecj0 = _sc_gather_vec(vec.reshape(N, VDIM * H), src_pad)
    vecjT = _vecj(vecj0, s1, s2, d_ij)
    xagg, vaggT = _sc_scatter(m, vecjT, dst_pad)

    vecagg = vaggT[:, :N, :].transpose(1, 0, 2)
    dx, dvec = _final(xagg[:N], vsum, vec3, vecagg, o_proj_W, o_proj_b)
    return (dx, dvec)


# speedup vs baseline: 6.4904x; 1.2566x over previous
import functools

import jax
import jax.numpy as jnp
from jax import lax
from jax.experimental import pallas as pl
from jax.experimental.pallas import tpu as pltpu
from jax.experimental.pallas import tpu_sc as plsc

N = 10000
E = 160000
H = 128
NH = 8
HD = H // NH
B = 100
NPS = N // B
VDIM = 8
CUTOFF = 5.0

NC = 2
NS = 16
NW = NC * NS

N_OUT = 10240
ACC2 = 10496
TRASH = N_OUT
ZR = ACC2 // NS
WR = N_OUT // NS

EPT = 10240
E_PAD = EPT * NS
GB = 128
NBB = EPT // GB
GPW = E_PAD // NW
GCH = 128
GCH2 = 40


def _silu(x):
    return x * jax.nn.sigmoid(x)


def _attn_kernel(x_ref, lnw_ref, lnb_ref, wq_ref, wk_ref, wv_ref, v_ref):
    xb = x_ref[0]
    m = jnp.mean(xb, axis=-1, keepdims=True)
    var = jnp.mean((xb - m) ** 2, axis=-1, keepdims=True)
    xb = (xb - m) / jnp.sqrt(var + 1e-5) * lnw_ref[0] + lnb_ref[0]
    q = jnp.dot(xb, wq_ref[...], preferred_element_type=jnp.float32)
    k = jnp.dot(xb, wk_ref[...], preferred_element_type=jnp.float32)
    vv = jnp.dot(xb, wv_ref[...], preferred_element_type=jnp.float32)
    scale = 1.0 / (HD ** 0.5)
    for h in range(NH):
        qh = q[:, h * HD:(h + 1) * HD]
        kh = k[:, h * HD:(h + 1) * HD]
        vh = vv[:, h * HD:(h + 1) * HD]
        sc = lax.dot_general(qh, kh, (((1,), (1,)), ((), ())),
                             preferred_element_type=jnp.float32) * scale
        sc = sc - jnp.max(sc, axis=-1, keepdims=True)
        ex = jnp.exp(sc)
        at = ex / jnp.sum(ex, axis=-1, keepdims=True)
        v3 = lax.dot_general(at, vh, (((1,), (0,)), ((), ())),
                             preferred_element_type=jnp.float32)
        v_ref[0, :, h * HD:(h + 1) * HD] = v3 * (1.0 / NPS)


def _attention(x, ln_w, ln_b, Wq, Wk, Wv):
    xg = x.reshape(B, NPS, H)
    out = pl.pallas_call(
        _attn_kernel,
        grid=(B,),
        in_specs=[
            pl.BlockSpec((1, NPS, H), lambda b: (b, 0, 0)),
            pl.BlockSpec((1, H), lambda b: (0, 0)),
            pl.BlockSpec((1, H), lambda b: (0, 0)),
            pl.BlockSpec((H, H), lambda b: (0, 0)),
            pl.BlockSpec((H, H), lambda b: (0, 0)),
            pl.BlockSpec((H, H), lambda b: (0, 0)),
        ],
        out_specs=pl.BlockSpec((1, NPS, H), lambda b: (b, 0, 0)),
        out_shape=jax.ShapeDtypeStruct((B, NPS, H), jnp.float32),
    )(xg, ln_w.reshape(1, H), ln_b.reshape(1, H), Wq, Wk, Wv)
    return out.reshape(N, H)


def _vec_kernel(vec_ref, w1_ref, w3_ref, vec3_ref, vsum_ref):
    acc = jnp.zeros((vec_ref.shape[1], H), jnp.float32)
    for d in range(VDIM):
        vd = vec_ref[0, :, d, :]
        acc = acc + vd
        vec3_ref[0, :, d, :] = jnp.dot(vd, w3_ref[...],
                                       preferred_element_type=jnp.float32)
    vsum_ref[0] = jnp.dot(acc, w1_ref[...], preferred_element_type=jnp.float32)


def _vec_proj(vec, vec_proj_W, vec_ln_w):
    wf = vec_proj_W * vec_ln_w[:, None]
    w1 = wf[:, :H]
    w3 = wf[:, H:]
    CH = 1000
    nch = N // CH
    vec3, vsum = pl.pallas_call(
        _vec_kernel,
        grid=(nch,),
        in_specs=[
            pl.BlockSpec((1, CH, VDIM, H), lambda i: (i, 0, 0, 0)),
            pl.BlockSpec((H, H), lambda i: (0, 0)),
            pl.BlockSpec((H, H), lambda i: (0, 0)),
        ],
        out_specs=[
            pl.BlockSpec((1, CH, VDIM, H), lambda i: (i, 0, 0, 0)),
            pl.BlockSpec((1, CH, H), lambda i: (i, 0, 0)),
        ],
        out_shape=[
            jax.ShapeDtypeStruct((nch, CH, VDIM, H), jnp.float32),
            jax.ShapeDtypeStruct((nch, CH, H), jnp.float32),
        ],
    )(vec.reshape(nch, CH, VDIM, H), w1, w3)
    return vec3.reshape(N, VDIM, H), vsum.reshape(N, H)


def _dv_kernel(f_ref, r_ref, w_ref, b_ref, dv_ref):
    pre = jnp.dot(f_ref[...], w_ref[...],
                  preferred_element_type=jnp.float32) + b_ref[0]
    r = r_ref[...]
    cut = 0.5 * (jnp.cos(jnp.pi * r / CUTOFF) + 1.0)
    cut = jnp.where(r < CUTOFF, cut, 0.0)
    dv_ref[...] = _silu(pre) * cut


def _dv(f_pad, r_pad, dv_proj_W, dv_proj_b):
    CH = 4096
    nch = E_PAD // CH
    return pl.pallas_call(
        _dv_kernel,
        grid=(nch,),
        in_specs=[
            pl.BlockSpec((CH, H), lambda i: (i, 0)),
            pl.BlockSpec((CH, 1), lambda i: (i, 0)),
            pl.BlockSpec((H, H), lambda i: (0, 0)),
            pl.BlockSpec((1, H), lambda i: (0, 0)),
        ],
        out_specs=pl.BlockSpec((CH, H), lambda i: (i, 0)),
        out_shape=jax.ShapeDtypeStruct((E_PAD, H), jnp.float32),
    )(f_pad, r_pad, dv_proj_W, dv_proj_b.reshape(1, H))


def _gather_db_body(tab_hbm, src3_hbm, out4_hbm, idx_v, rows_v, gsem, wsem,
                    n2):
    wid = lax.axis_index("c") * NS + lax.axis_index("s")

    def pair(ch2, carry):
        @pl.when(ch2 > 0)
        def _():
            for k in (0, 1):
                pltpu.make_async_copy(rows_v.at[k], out4_hbm.at[wid, 0],
                                      wsem).wait()
        cps = []
        for k in (0, 1):
            pltpu.sync_copy(src3_hbm.at[wid, ch2 * 2 + k], idx_v.at[k])
            cps.append(pltpu.async_copy(tab_hbm.at[idx_v.at[k]], rows_v.at[k],
                                        gsem))
        for k in (0, 1):
            cps[k].wait()
            pltpu.async_copy(rows_v.at[k], out4_hbm.at[wid, ch2 * 2 + k], wsem)
        return carry

    lax.fori_loop(0, n2, pair, jnp.int32(0))
    for k in (0, 1):
        pltpu.make_async_copy(rows_v.at[k], out4_hbm.at[wid, 0], wsem).wait()


def _gather_rows(table, src_pad, width, chunk):
    mesh = plsc.VectorSubcoreMesh(core_axis_name="c", subcore_axis_name="s")
    n_chunk = GPW // chunk
    body = functools.partial(_gather_db_body, n2=n_chunk // 2)
    f = pl.kernel(
        body,
        out_type=jax.ShapeDtypeStruct((NW, n_chunk, chunk, width), jnp.float32),
        mesh=mesh,
        scratch_types=[
            pltpu.VMEM((2, chunk), jnp.int32),
            pltpu.VMEM((2, chunk, width), jnp.float32),
            pltpu.SemaphoreType.DMA,
            pltpu.SemaphoreType.DMA,
        ],
    )
    out = f(table, src_pad.reshape(NW, n_chunk, chunk))
    return out.reshape(E_PAD, width)


def _sc_gather(v, src_pad):
    return _gather_rows(v, src_pad, H, GCH)


def _edge_kernel(vj_ref, dv_ref, w_ref, b_ref, lnv_ref, m_ref, s1_ref, s2_ref):
    m = vj_ref[...] * dv_ref[...]
    s = _silu(jnp.dot(m, w_ref[...],
                      preferred_element_type=jnp.float32) + b_ref[0])
    m_ref[...] = m
    s1_ref[...] = s[:, :H] * lnv_ref[0]
    s2_ref[...] = s[:, H:]


def _edge(vj0, dvp, s_proj_W, s_proj_b, vec_ln_w):
    CH = 4096
    nch = E_PAD // CH
    return pl.pallas_call(
        _edge_kernel,
        grid=(nch,),
        in_specs=[
            pl.BlockSpec((CH, H), lambda i: (i, 0)),
            pl.BlockSpec((CH, H), lambda i: (i, 0)),
            pl.BlockSpec((H, 2 * H), lambda i: (0, 0)),
            pl.BlockSpec((1, 2 * H), lambda i: (0, 0)),
            pl.BlockSpec((1, H), lambda i: (0, 0)),
        ],
        out_specs=[
            pl.BlockSpec((CH, H), lambda i: (i, 0)),
            pl.BlockSpec((CH, H), lambda i: (i, 0)),
            pl.BlockSpec((CH, H), lambda i: (i, 0)),
        ],
        out_shape=[
            jax.ShapeDtypeStruct((E_PAD, H), jnp.float32),
            jax.ShapeDtypeStruct((E_PAD, H), jnp.float32),
            jax.ShapeDtypeStruct((E_PAD, H), jnp.float32),
        ],
    )(vj0, dvp, s_proj_W, s_proj_b.reshape(1, 2 * H), vec_ln_w.reshape(1, H))


def _sc_gather_vec(vec2d, src_pad):
    return _gather_rows(vec2d, src_pad, VDIM * H, GCH2)


def _vecj_kernel(vj0_ref, s1_ref, s2_ref, dp_ref, out_ref):
    s1 = s1_ref[...]
    s2 = s2_ref[...]
    for d in range(VDIM):
        out_ref[d] = (vj0_ref[:, d * H:(d + 1) * H] * s1
                      + s2 * dp_ref[:, d:d + 1])


def _vecj(vecj0, s1, s2, d_ij):
    CHE = 1024
    nch = E_PAD // CHE
    dp = jnp.pad(d_ij, ((0, E_PAD - E), (0, 0)))
    return pl.pallas_call(
        _vecj_kernel,
        grid=(nch,),
        in_specs=[
            pl.BlockSpec((CHE, VDIM * H), lambda i: (i, 0)),
            pl.BlockSpec((CHE, H), lambda i: (i, 0)),
            pl.BlockSpec((CHE, H), lambda i: (i, 0)),
            pl.BlockSpec((CHE, VDIM), lambda i: (i, 0)),
        ],
        out_specs=pl.BlockSpec((VDIM, CHE, H), lambda i: (0, i, 0)),
        out_shape=jax.ShapeDtypeStruct((VDIM, E_PAD, H), jnp.float32),
    )(vecj0, s1, s2, dp)


def _sc_scatter_body(m_hbm, vjT_hbm, dst_hbm, z_hbm, xagg_hbm, vagg_hbm,
                     dstraw, idxbuf, databuf, acc):
    c = lax.axis_index("c")
    sid = lax.axis_index("s")

    def do_slab(src4d, slab_out):
        pltpu.sync_copy(z_hbm, acc.at[pl.ds(sid * ZR, ZR)])
        plsc.subcore_barrier()

        def batch(b, carry):
            pltpu.sync_copy(dst_hbm.at[sid, b], dstraw)
            for j in range(GB // 16):
                d16 = dstraw[pl.ds(j * 16, 16)]
                idxbuf[pl.ds(j * 16, 16)] = jnp.where(d16 < 0, TRASH, d16)
            pltpu.sync_copy(src4d.at[sid, b], databuf)
            pltpu.sync_copy(databuf, acc.at[idxbuf], add=True)
            return carry

        lax.fori_loop(0, NBB, batch, jnp.int32(0))
        plsc.subcore_barrier()
        pltpu.sync_copy(acc.at[pl.ds(sid * WR, WR)],
                        slab_out.at[pl.ds(sid * WR, WR)])
        plsc.subcore_barrier()

    @pl.when(c == 0)
    def _():
        do_slab(m_hbm, xagg_hbm)
        for d in (0, 1, 2):
            do_slab(vjT_hbm.at[d], vagg_hbm.at[d])

    @pl.when(c == 1)
    def _():
        for d in (3, 4, 5, 6, 7):
            do_slab(vjT_hbm.at[d], vagg_hbm.at[d])


def _sc_scatter(m, vecjT, dst_pad):
    mesh = plsc.VectorSubcoreMesh(core_axis_name="c", subcore_axis_name="s")
    f = pl.kernel(
        _sc_scatter_body,
        out_type=[
            jax.ShapeDtypeStruct((N_OUT, H), jnp.float32),
            jax.ShapeDtypeStruct((VDIM, N_OUT, H), jnp.float32),
        ],
        mesh=mesh,
        scratch_types=[
            pltpu.VMEM((GB,), jnp.int32),
            pltpu.VMEM((GB,), jnp.int32),
            pltpu.VMEM((GB, H), jnp.float32),
            pltpu.VMEM_SHARED((ACC2, H), jnp.float32),
        ],
    )
    z = jnp.zeros((ZR, H), jnp.float32)
    xagg, vaggT = f(m.reshape(NS, NBB, GB, H),
                    vecjT.reshape(VDIM, NS, NBB, GB, H),
                    dst_pad.reshape(NS, NBB, GB), z)
    return xagg, vaggT


def _final_kernel(xagg_ref, vsum_ref, vec3_ref, vagg_ref, w_ref, b_ref,
                  dx_ref, dvec_ref):
    o = jnp.dot(xagg_ref[0], w_ref[...],
                preferred_element_type=jnp.float32) + b_ref[0]
    o1 = o[:, :H]
    o2 = o[:, H:2 * H]
    o3 = o[:, 2 * H:]
    dx_ref[0] = vsum_ref[0] * o2 + o3
    for d in range(VDIM):
        dvec_ref[0, :, d, :] = vec3_ref[0, :, d, :] * o1 + vagg_ref[0, :, d, :]


def _final(xagg, vsum, vec3, vagg, o_proj_W, o_proj_b):
    CH = 1000
    nch = N // CH
    dx, dvec = pl.pallas_call(
        _final_kernel,
        grid=(nch,),
        in_specs=[
            pl.BlockSpec((1, CH, H), lambda i: (i, 0, 0)),
            pl.BlockSpec((1, CH, H), lambda i: (i, 0, 0)),
            pl.BlockSpec((1, CH, VDIM, H), lambda i: (i, 0, 0, 0)),
            pl.BlockSpec((1, CH, VDIM, H), lambda i: (i, 0, 0, 0)),
            pl.BlockSpec((H, 3 * H), lambda i: (0, 0)),
            pl.BlockSpec((1, 3 * H), lambda i: (0, 0)),
        ],
        out_specs=[
            pl.BlockSpec((1, CH, H), lambda i: (i, 0, 0)),
            pl.BlockSpec((1, CH, VDIM, H), lambda i: (i, 0, 0, 0)),
        ],
        out_shape=[
            jax.ShapeDtypeStruct((nch, CH, H), jnp.float32),
            jax.ShapeDtypeStruct((nch, CH, VDIM, H), jnp.float32),
        ],
    )(xagg.reshape(nch, CH, H), vsum.reshape(nch, CH, H),
      vec3.reshape(nch, CH, VDIM, H), vagg.reshape(nch, CH, VDIM, H),
      o_proj_W, o_proj_b.reshape(1, 3 * H))
    return dx.reshape(N, H), dvec.reshape(N, VDIM, H)


def kernel(x, vec, edge_index, r_ij, f_ij, d_ij, batch, ln_w, ln_b, vec_ln_w,
           Wq, Wk, Wv, vec_proj_W, dv_proj_W, dv_proj_b,
           s_proj_W, s_proj_b, o_proj_W, o_proj_b):
    del batch

    src = edge_index[0]
    dst = edge_index[1]
    pad_e = E_PAD - E
    src_pad = jnp.pad(src, (0, pad_e))
    dst_pad = jnp.pad(dst, (0, pad_e), constant_values=-1)
    f_pad = jnp.pad(f_ij, ((0, pad_e), (0, 0)))
    r_pad = jnp.pad(r_ij, (0, pad_e), constant_values=2.0 * CUTOFF)

    v = _attention(x, ln_w, ln_b, Wq, Wk, Wv)
    vec3, vsum = _vec_proj(vec, vec_proj_W, vec_ln_w)
    dvp = _dv(f_pad, r_pad.reshape(E_PAD, 1), dv_proj_W, dv_proj_b)

    vj0 = _sc_gather(v, src_pad)
    m, s1, s2 = _edge(vj0, dvp, s_proj_W, s_proj_b, vec_ln_w)

    vecj0 = _sc_gather_vec(vec.reshape(N, VDIM * H), src_pad)
    vecjT = _vecj(vecj0, s1, s2, d_ij)
    xagg, vaggT = _sc_scatter(m, vecjT, dst_pad)

    vecagg = vaggT[:, :N, :].transpose(1, 0, 2)
    dx, dvec = _final(xagg[:N], vsum, vec3, vecagg, o_proj_W, o_proj_b)
    return (dx, dvec)


# speedup vs baseline: 7.7209x; 1.1896x over previous
import functools

import jax
import jax.numpy as jnp
from jax import lax
from jax.experimental import pallas as pl
from jax.experimental.pallas import tpu as pltpu
from jax.experimental.pallas import tpu_sc as plsc

N = 10000
E = 160000
H = 128
NH = 8
HD = H // NH
B = 100
NPS = N // B
VDIM = 8
CUTOFF = 5.0

NC = 2
NS = 16
NW = NC * NS

N_OUT = 10240
ACC2 = 10496
TRASH = N_OUT
ZR = ACC2 // NS
WR = N_OUT // NS

EPT = 10240
E_PAD = EPT * NS
GB = 128
NBB = EPT // GB
GPW = E_PAD // NW
GCH = 128
GCH2 = 40


def _silu(x):
    return x * jax.nn.sigmoid(x)


def _attn_kernel(x_ref, lnw_ref, lnb_ref, wq_ref, wk_ref, wv_ref, v_ref):
    xb = x_ref[0]
    m = jnp.mean(xb, axis=-1, keepdims=True)
    var = jnp.mean((xb - m) ** 2, axis=-1, keepdims=True)
    xb = (xb - m) / jnp.sqrt(var + 1e-5) * lnw_ref[0] + lnb_ref[0]
    q = jnp.dot(xb, wq_ref[...], preferred_element_type=jnp.float32)
    k = jnp.dot(xb, wk_ref[...], preferred_element_type=jnp.float32)
    vv = jnp.dot(xb, wv_ref[...], preferred_element_type=jnp.float32)
    scale = 1.0 / (HD ** 0.5)
    for h in range(NH):
        qh = q[:, h * HD:(h + 1) * HD]
        kh = k[:, h * HD:(h + 1) * HD]
        vh = vv[:, h * HD:(h + 1) * HD]
        sc = lax.dot_general(qh, kh, (((1,), (1,)), ((), ())),
                             preferred_element_type=jnp.float32) * scale
        sc = sc - jnp.max(sc, axis=-1, keepdims=True)
        ex = jnp.exp(sc)
        at = ex / jnp.sum(ex, axis=-1, keepdims=True)
        v3 = lax.dot_general(at, vh, (((1,), (0,)), ((), ())),
                             preferred_element_type=jnp.float32)
        v_ref[0, :, h * HD:(h + 1) * HD] = v3 * (1.0 / NPS)


def _attention(x, ln_w, ln_b, Wq, Wk, Wv):
    xg = x.reshape(B, NPS, H)
    out = pl.pallas_call(
        _attn_kernel,
        grid=(B,),
        in_specs=[
            pl.BlockSpec((1, NPS, H), lambda b: (b, 0, 0)),
            pl.BlockSpec((1, H), lambda b: (0, 0)),
            pl.BlockSpec((1, H), lambda b: (0, 0)),
            pl.BlockSpec((H, H), lambda b: (0, 0)),
            pl.BlockSpec((H, H), lambda b: (0, 0)),
            pl.BlockSpec((H, H), lambda b: (0, 0)),
        ],
        out_specs=pl.BlockSpec((1, NPS, H), lambda b: (b, 0, 0)),
        out_shape=jax.ShapeDtypeStruct((B, NPS, H), jnp.float32),
    )(xg, ln_w.reshape(1, H), ln_b.reshape(1, H), Wq, Wk, Wv)
    return out.reshape(N, H)


def _vec_kernel(vec_ref, w1_ref, w3_ref, vec3_ref, vsum_ref):
    acc = jnp.zeros((vec_ref.shape[1], H), jnp.float32)
    for d in range(VDIM):
        vd = vec_ref[0, :, d, :]
        acc = acc + vd
        vec3_ref[0, :, d, :] = jnp.dot(vd, w3_ref[...],
                                       preferred_element_type=jnp.float32)
    vsum_ref[0] = jnp.dot(acc, w1_ref[...], preferred_element_type=jnp.float32)


def _vec_proj(vec, vec_proj_W, vec_ln_w):
    wf = vec_proj_W * vec_ln_w[:, None]
    w1 = wf[:, :H]
    w3 = wf[:, H:]
    CH = 1000
    nch = N // CH
    vec3, vsum = pl.pallas_call(
        _vec_kernel,
        grid=(nch,),
        in_specs=[
            pl.BlockSpec((1, CH, VDIM, H), lambda i: (i, 0, 0, 0)),
            pl.BlockSpec((H, H), lambda i: (0, 0)),
            pl.BlockSpec((H, H), lambda i: (0, 0)),
        ],
        out_specs=[
            pl.BlockSpec((1, CH, VDIM, H), lambda i: (i, 0, 0, 0)),
            pl.BlockSpec((1, CH, H), lambda i: (i, 0, 0)),
        ],
        out_shape=[
            jax.ShapeDtypeStruct((nch, CH, VDIM, H), jnp.float32),
            jax.ShapeDtypeStruct((nch, CH, H), jnp.float32),
        ],
    )(vec.reshape(nch, CH, VDIM, H), w1, w3)
    return vec3.reshape(N, VDIM, H), vsum.reshape(N, H)


def _dv_kernel(f_ref, r_ref, w_ref, b_ref, dv_ref):
    pre = jnp.dot(f_ref[...], w_ref[...],
                  preferred_element_type=jnp.float32) + b_ref[0]
    r = r_ref[...]
    cut = 0.5 * (jnp.cos(jnp.pi * r / CUTOFF) + 1.0)
    cut = jnp.where(r < CUTOFF, cut, 0.0)
    dv_ref[...] = _silu(pre) * cut


def _dv(f_pad, r_pad, dv_proj_W, dv_proj_b):
    CH = 4096
    nch = E_PAD // CH
    return pl.pallas_call(
        _dv_kernel,
        grid=(nch,),
        in_specs=[
            pl.BlockSpec((CH, H), lambda i: (i, 0)),
            pl.BlockSpec((CH, 1), lambda i: (i, 0)),
            pl.BlockSpec((H, H), lambda i: (0, 0)),
            pl.BlockSpec((1, H), lambda i: (0, 0)),
        ],
        out_specs=pl.BlockSpec((CH, H), lambda i: (i, 0)),
        out_shape=jax.ShapeDtypeStruct((E_PAD, H), jnp.float32),
    )(f_pad, r_pad, dv_proj_W, dv_proj_b.reshape(1, H))


def _gather_db_body(tab_hbm, src3_hbm, out4_hbm, idx_v, rows_v, gsem, wsem,
                    n2):
    wid = lax.axis_index("c") * NS + lax.axis_index("s")

    def pair(ch2, carry):
        @pl.when(ch2 > 0)
        def _():
            for k in (0, 1):
                pltpu.make_async_copy(rows_v.at[k], out4_hbm.at[wid, 0],
                                      wsem).wait()
        cps = []
        for k in (0, 1):
            pltpu.sync_copy(src3_hbm.at[wid, ch2 * 2 + k], idx_v.at[k])
            cps.append(pltpu.async_copy(tab_hbm.at[idx_v.at[k]], rows_v.at[k],
                                        gsem))
        for k in (0, 1):
            cps[k].wait()
            pltpu.async_copy(rows_v.at[k], out4_hbm.at[wid, ch2 * 2 + k], wsem)
        return carry

    lax.fori_loop(0, n2, pair, jnp.int32(0))
    for k in (0, 1):
        pltpu.make_async_copy(rows_v.at[k], out4_hbm.at[wid, 0], wsem).wait()


def _gather_rows(table, src_pad, width, chunk):
    mesh = plsc.VectorSubcoreMesh(core_axis_name="c", subcore_axis_name="s")
    n_chunk = GPW // chunk
    body = functools.partial(_gather_db_body, n2=n_chunk // 2)
    f = pl.kernel(
        body,
        out_type=jax.ShapeDtypeStruct((NW, n_chunk, chunk, width), jnp.float32),
        mesh=mesh,
        scratch_types=[
            pltpu.VMEM((2, chunk), jnp.int32),
            pltpu.VMEM((2, chunk, width), jnp.float32),
            pltpu.SemaphoreType.DMA,
            pltpu.SemaphoreType.DMA,
        ],
    )
    out = f(table, src_pad.reshape(NW, n_chunk, chunk))
    return out.reshape(E_PAD, width)


def _sc_gather(v, src_pad):
    return _gather_rows(v, src_pad, H, GCH)


def _edge_kernel(vj_ref, dv_ref, w_ref, b_ref, lnv_ref, m_ref, s1_ref, s2_ref):
    m = vj_ref[...] * dv_ref[...]
    s = _silu(jnp.dot(m, w_ref[...],
                      preferred_element_type=jnp.float32) + b_ref[0])
    m_ref[...] = m
    s1_ref[...] = s[:, :H] * lnv_ref[0]
    s2_ref[...] = s[:, H:]


def _edge(vj0, dvp, s_proj_W, s_proj_b, vec_ln_w):
    CH = 4096
    nch = E_PAD // CH
    return pl.pallas_call(
        _edge_kernel,
        grid=(nch,),
        in_specs=[
            pl.BlockSpec((CH, H), lambda i: (i, 0)),
            pl.BlockSpec((CH, H), lambda i: (i, 0)),
            pl.BlockSpec((H, 2 * H), lambda i: (0, 0)),
            pl.BlockSpec((1, 2 * H), lambda i: (0, 0)),
            pl.BlockSpec((1, H), lambda i: (0, 0)),
        ],
        out_specs=[
            pl.BlockSpec((CH, H), lambda i: (i, 0)),
            pl.BlockSpec((CH, H), lambda i: (i, 0)),
            pl.BlockSpec((CH, H), lambda i: (i, 0)),
        ],
        out_shape=[
            jax.ShapeDtypeStruct((E_PAD, H), jnp.float32),
            jax.ShapeDtypeStruct((E_PAD, H), jnp.float32),
            jax.ShapeDtypeStruct((E_PAD, H), jnp.float32),
        ],
    )(vj0, dvp, s_proj_W, s_proj_b.reshape(1, 2 * H), vec_ln_w.reshape(1, H))


def _sc_gather_vec(vec2d, src_pad):
    return _gather_rows(vec2d, src_pad, VDIM * H, GCH2)


def _vecj_kernel(vj0_ref, s1_ref, s2_ref, dp_ref, out_ref):
    s1 = s1_ref[...]
    s2 = s2_ref[...]
    for d in range(VDIM):
        out_ref[d] = (vj0_ref[:, d * H:(d + 1) * H] * s1
                      + s2 * dp_ref[:, d:d + 1])


def _vecj(vecj0, s1, s2, d_ij):
    CHE = 1024
    nch = E_PAD // CHE
    dp = jnp.pad(d_ij, ((0, E_PAD - E), (0, 0)))
    return pl.pallas_call(
        _vecj_kernel,
        grid=(nch,),
        in_specs=[
            pl.BlockSpec((CHE, VDIM * H), lambda i: (i, 0)),
            pl.BlockSpec((CHE, H), lambda i: (i, 0)),
            pl.BlockSpec((CHE, H), lambda i: (i, 0)),
            pl.BlockSpec((CHE, VDIM), lambda i: (i, 0)),
        ],
        out_specs=pl.BlockSpec((VDIM, CHE, H), lambda i: (0, i, 0)),
        out_shape=jax.ShapeDtypeStruct((VDIM, E_PAD, H), jnp.float32),
    )(vecj0, s1, s2, dp)


def _sc_scatter_body(m_hbm, vjT_hbm, dst_hbm, z_hbm, xagg_hbm, vagg_hbm,
                     dstraw, idxbuf, databuf, acc, dsem, datsem):
    c = lax.axis_index("c")
    sid = lax.axis_index("s")

    def do_slab(src4d, slab_out):
        pltpu.sync_copy(z_hbm, acc.at[pl.ds(sid * ZR, ZR)])
        plsc.subcore_barrier()

        for k in (0, 1):
            pltpu.async_copy(dst_hbm.at[sid, k], dstraw.at[k], dsem)
            pltpu.async_copy(src4d.at[sid, k], databuf.at[k], datsem)

        def pair(b2, carry):
            for k in (0, 1):
                b = b2 * 2 + k
                pltpu.make_async_copy(dst_hbm.at[sid, 0], dstraw.at[k],
                                      dsem).wait()
                pltpu.make_async_copy(src4d.at[sid, 0], databuf.at[k],
                                      datsem).wait()
                for j in range(GB // 16):
                    d16 = dstraw[k, pl.ds(j * 16, 16)]
                    idxbuf[pl.ds(j * 16, 16)] = jnp.where(d16 < 0, TRASH, d16)
                pltpu.sync_copy(databuf.at[k], acc.at[idxbuf], add=True)

                @pl.when(b + 2 < NBB)
                def _():
                    pltpu.async_copy(dst_hbm.at[sid, b + 2], dstraw.at[k], dsem)
                    pltpu.async_copy(src4d.at[sid, b + 2], databuf.at[k],
                                     datsem)
            return carry

        lax.fori_loop(0, NBB // 2, pair, jnp.int32(0))
        plsc.subcore_barrier()
        pltpu.sync_copy(acc.at[pl.ds(sid * WR, WR)],
                        slab_out.at[pl.ds(sid * WR, WR)])
        plsc.subcore_barrier()

    @pl.when(c == 0)
    def _():
        do_slab(m_hbm, xagg_hbm)
        for d in (0, 1, 2):
            do_slab(vjT_hbm.at[d], vagg_hbm.at[d])

    @pl.when(c == 1)
    def _():
        for d in (3, 4, 5, 6, 7):
            do_slab(vjT_hbm.at[d], vagg_hbm.at[d])


def _sc_scatter(m, vecjT, dst_pad):
    mesh = plsc.VectorSubcoreMesh(core_axis_name="c", subcore_axis_name="s")
    f = pl.kernel(
        _sc_scatter_body,
        out_type=[
            jax.ShapeDtypeStruct((N_OUT, H), jnp.float32),
            jax.ShapeDtypeStruct((VDIM, N_OUT, H), jnp.float32),
        ],
        mesh=mesh,
        scratch_types=[
            pltpu.VMEM((2, GB), jnp.int32),
            pltpu.VMEM((GB,), jnp.int32),
            pltpu.VMEM((2, GB, H), jnp.float32),
            pltpu.VMEM_SHARED((ACC2, H), jnp.float32),
            pltpu.SemaphoreType.DMA,
            pltpu.SemaphoreType.DMA,
        ],
    )
    z = jnp.zeros((ZR, H), jnp.float32)
    xagg, vaggT = f(m.reshape(NS, NBB, GB, H),
                    vecjT.reshape(VDIM, NS, NBB, GB, H),
                    dst_pad.reshape(NS, NBB, GB), z)
    return xagg, vaggT


def _final_kernel(xagg_ref, vsum_ref, vec3_ref, vagg_ref, w_ref, b_ref,
                  dx_ref, dvec_ref):
    o = jnp.dot(xagg_ref[0], w_ref[...],
                preferred_element_type=jnp.float32) + b_ref[0]
    o1 = o[:, :H]
    o2 = o[:, H:2 * H]
    o3 = o[:, 2 * H:]
    dx_ref[0] = vsum_ref[0] * o2 + o3
    for d in range(VDIM):
        dvec_ref[0, :, d, :] = vec3_ref[0, :, d, :] * o1 + vagg_ref[0, :, d, :]


def _final(xagg, vsum, vec3, vagg, o_proj_W, o_proj_b):
    CH = 1000
    nch = N // CH
    dx, dvec = pl.pallas_call(
        _final_kernel,
        grid=(nch,),
        in_specs=[
            pl.BlockSpec((1, CH, H), lambda i: (i, 0, 0)),
            pl.BlockSpec((1, CH, H), lambda i: (i, 0, 0)),
            pl.BlockSpec((1, CH, VDIM, H), lambda i: (i, 0, 0, 0)),
            pl.BlockSpec((1, CH, VDIM, H), lambda i: (i, 0, 0, 0)),
            pl.BlockSpec((H, 3 * H), lambda i: (0, 0)),
            pl.BlockSpec((1, 3 * H), lambda i: (0, 0)),
        ],
        out_specs=[
            pl.BlockSpec((1, CH, H), lambda i: (i, 0, 0)),
            pl.BlockSpec((1, CH, VDIM, H), lambda i: (i, 0, 0, 0)),
        ],
        out_shape=[
            jax.ShapeDtypeStruct((nch, CH, H), jnp.float32),
            jax.ShapeDtypeStruct((nch, CH, VDIM, H), jnp.float32),
        ],
    )(xagg.reshape(nch, CH, H), vsum.reshape(nch, CH, H),
      vec3.reshape(nch, CH, VDIM, H), vagg.reshape(nch, CH, VDIM, H),
      o_proj_W, o_proj_b.reshape(1, 3 * H))
    return dx.reshape(N, H), dvec.reshape(N, VDIM, H)


def kernel(x, vec, edge_index, r_ij, f_ij, d_ij, batch, ln_w, ln_b, vec_ln_w,
           Wq, Wk, Wv, vec_proj_W, dv_proj_W, dv_proj_b,
           s_proj_W, s_proj_b, o_proj_W, o_proj_b):
    del batch

    src = edge_index[0]
    dst = edge_index[1]
    pad_e = E_PAD - E
    src_pad = jnp.pad(src, (0, pad_e))
    dst_pad = jnp.pad(dst, (0, pad_e), constant_values=-1)
    f_pad = jnp.pad(f_ij, ((0, pad_e), (0, 0)))
    r_pad = jnp.pad(r_ij, (0, pad_e), constant_values=2.0 * CUTOFF)

    v = _attention(x, ln_w, ln_b, Wq, Wk, Wv)
    vec3, vsum = _vec_proj(vec, vec_proj_W, vec_ln_w)
    dvp = _dv(f_pad, r_pad.reshape(E_PAD, 1), dv_proj_W, dv_proj_b)

    vj0 = _sc_gather(v, src_pad)
    m, s1, s2 = _edge(vj0, dvp, s_proj_W, s_proj_b, vec_ln_w)

    vecj0 = _sc_gather_vec(vec.reshape(N, VDIM * H), src_pad)
    vecjT = _vecj(vecj0, s1, s2, d_ij)
    xagg, vaggT = _sc_scatter(m, vecjT, dst_pad)

    vecagg = vaggT[:, :N, :].transpose(1, 0, 2)
    dx, dvec = _final(xagg[:N], vsum, vec3, vecagg, o_proj_W, o_proj_b)
    return (dx, dvec)
